# Initial kernel scaffold; baseline (speedup 1.0000x reference)
import jax
import jax.numpy as jnp
from jax.experimental import pallas as pl


def kernel(x, edge_index, edge_weight, W1, b1, W2, b2, Wroot, bconv, Wh, bh):
    raise NotImplementedError("write your pallas kernel here")


# speedup vs baseline: 3.5578x; 3.5578x over previous
import functools

import jax
import jax.numpy as jnp
from jax import lax
from jax.experimental import pallas as pl
from jax.experimental.pallas import tpu as pltpu
from jax.experimental.pallas import tpu_sc as plsc

N = 10000
E = 160000
D_IN = 128
HID = 16
K = 32
ROW = K * HID + HID
SROW = 32
NC = 2
NS = 16
NW = NC * NS
C = 64
CHUNKS = E // C
BASE_CH = CHUNKS // NW
EXTRA = CHUNKS - BASE_CH * NW
ZR = 624
MBLK = 1000


def _matmul_body(x_ref, w_ref, o_ref):
    o_ref[...] = jnp.dot(x_ref[...], w_ref[...],
                         preferred_element_type=jnp.float32)


def _precompute_q(x, w2aug):
    return pl.pallas_call(
        _matmul_body,
        grid=(N // MBLK,),
        in_specs=[
            pl.BlockSpec((MBLK, D_IN), lambda i: (i, 0)),
            pl.BlockSpec((D_IN, ROW), lambda i: (0, 0)),
        ],
        out_specs=pl.BlockSpec((MBLK, ROW), lambda i: (i, 0)),
        out_shape=jax.ShapeDtypeStruct((N, ROW), jnp.float32),
    )(x, w2aug)


def _combine_body(parts_ref, x_ref, wroot_ref, bconv_ref, wh_ref, bh_ref,
                  o_ref):
    s = parts_ref[0] + parts_ref[1]
    msg = s[:, 0:HID]
    cnt = s[:, HID:HID + 1]
    agg = msg / jnp.maximum(cnt, 1.0)
    root = jnp.dot(x_ref[...], wroot_ref[...],
                   preferred_element_type=jnp.float32)
    h = jnp.maximum(agg + root + bconv_ref[...], 0.0)
    o_ref[...] = jnp.dot(h, wh_ref[...],
                         preferred_element_type=jnp.float32) + bh_ref[0, 0]


def _combine(parts, x, wroot, bconv2, wh, bh2):
    return pl.pallas_call(
        _combine_body,
        grid=(N // MBLK,),
        in_specs=[
            pl.BlockSpec((NC, MBLK, SROW), lambda i: (0, i, 0)),
            pl.BlockSpec((MBLK, D_IN), lambda i: (i, 0)),
            pl.BlockSpec((D_IN, HID), lambda i: (0, 0)),
            pl.BlockSpec((1, HID), lambda i: (0, 0)),
            pl.BlockSpec((HID, 1), lambda i: (0, 0)),
            pl.BlockSpec((1, 1), lambda i: (0, 0)),
        ],
        out_specs=pl.BlockSpec((MBLK, 1), lambda i: (i, 0)),
        out_shape=jax.ShapeDtypeStruct((N, 1), jnp.float32),
    )(parts, x, wroot, bconv2, wh, bh2)


def _sc_body(q_hbm, src_hbm, dst_hbm, ew_hbm, w1_hbm, b1_hbm, out_hbm,
             srcb, dstb, ewb, qrows, msgb, zb, w1v, b1v, acc_sh, sem):
    cid = lax.axis_index("c")
    sid = lax.axis_index("s")
    wid = sid * NC + cid

    zeros16 = jnp.zeros((HID,), jnp.float32)

    @pl.loop(0, ZR)
    def _zrow(r):
        zb[r, pl.ds(0, HID)] = zeros16
        zb[r, pl.ds(HID, HID)] = zeros16

    zstart = pl.multiple_of(sid * ZR, 8)
    pltpu.sync_copy(zb, acc_sh.at[pl.ds(zstart, ZR)])

    @pl.when(sid == NS - 1)
    def _ztail():
        pltpu.sync_copy(zb.at[pl.ds(0, N - NS * ZR)],
                        acc_sh.at[pl.ds(NS * ZR, N - NS * ZR)])

    lane = lax.iota(jnp.int32, HID)
    cvec = jnp.where(lane == 0, 1.0, 0.0).astype(jnp.float32)

    @pl.loop(0, C)
    def _minit(e):
        msgb[e, pl.ds(HID, HID)] = cvec

    pltpu.sync_copy(w1_hbm, w1v)
    pltpu.sync_copy(b1_hbm, b1v)
    w1lo, w1hi = w1v[pl.ds(0, 16)], w1v[pl.ds(16, 16)]
    b1lo, b1hi = b1v[pl.ds(0, 16)], b1v[pl.ds(16, 16)]
    w1s = [w1lo[i] for i in range(16)] + [w1hi[i] for i in range(16)]
    b1s = [b1lo[i] for i in range(16)] + [b1hi[i] for i in range(16)]

    plsc.subcore_barrier()

    nch = BASE_CH + jnp.where(wid < EXTRA, 1, 0)
    ch0 = wid * BASE_CH + jnp.minimum(wid, EXTRA)

    @pl.loop(0, nch)
    def _chunk(i):
        eb = pl.multiple_of((ch0 + i) * C, C)
        pltpu.sync_copy(src_hbm.at[pl.ds(eb, C)], srcb)
        pltpu.sync_copy(dst_hbm.at[pl.ds(eb, C)], dstb)
        pltpu.sync_copy(ew_hbm.at[pl.ds(eb, C)], ewb)
        pltpu.async_copy(q_hbm.at[srcb], qrows, sem).wait()

        @pl.loop(0, C // 16)
        def _group(g):
            gb = g * 16
            ewv = ewb[pl.ds(gb, 16)]
            hks = [jnp.maximum(ewv * w1s[k] + b1s[k], 0.0) for k in range(K)]
            for j in range(16):
                e = gb + j
                a = [qrows[e, pl.ds(K * HID, HID)],
                     hks[0][j] * qrows[e, pl.ds(0, HID)],
                     hks[1][j] * qrows[e, pl.ds(HID, HID)],
                     hks[2][j] * qrows[e, pl.ds(2 * HID, HID)]]
                for k in range(3, K):
                    a[(k + 1) % 4] = (a[(k + 1) % 4] + hks[k][j]
                                      * qrows[e, pl.ds(k * HID, HID)])
                msgb[e, pl.ds(0, HID)] = (a[0] + a[1]) + (a[2] + a[3])

        pltpu.sync_copy(msgb, acc_sh.at[dstb], add=True)

    plsc.subcore_barrier()
    pltpu.sync_copy(acc_sh.at[pl.ds(zstart, ZR)],
                    out_hbm.at[cid, pl.ds(zstart, ZR)])

    @pl.when(sid == NS - 1)
    def _wtail():
        pltpu.sync_copy(acc_sh.at[pl.ds(NS * ZR, N - NS * ZR)],
                        out_hbm.at[cid, pl.ds(NS * ZR, N - NS * ZR)])


_sc_edge = functools.partial(
    pl.kernel,
    out_type=jax.ShapeDtypeStruct((NC, N, SROW), jnp.float32),
    mesh=plsc.VectorSubcoreMesh(core_axis_name="c", subcore_axis_name="s",
                                num_cores=NC, num_subcores=NS),
    compiler_params=pltpu.CompilerParams(use_tc_tiling_on_sc=False),
    scratch_types=[
        pltpu.VMEM((C,), jnp.int32),
        pltpu.VMEM((C,), jnp.int32),
        pltpu.VMEM((C,), jnp.float32),
        pltpu.VMEM((C, ROW), jnp.float32),
        pltpu.VMEM((C, SROW), jnp.float32),
        pltpu.VMEM((ZR, SROW), jnp.float32),
        pltpu.VMEM((K,), jnp.float32),
        pltpu.VMEM((K,), jnp.float32),
        pltpu.VMEM_SHARED((N, SROW), jnp.float32),
        pltpu.SemaphoreType.DMA,
    ],
)(_sc_body)


def kernel(x, edge_index, edge_weight, W1, b1, W2, b2, Wroot, bconv, Wh, bh):
    w2t = W2.reshape(K, D_IN, HID).transpose(1, 0, 2).reshape(D_IN, K * HID)
    w2aug = jnp.concatenate([w2t, b2.reshape(D_IN, HID)], axis=1)

    q = _precompute_q(x, w2aug)
    parts = _sc_edge(q, edge_index[0], edge_index[1], edge_weight,
                     W1.reshape(K), b1)
    out = _combine(parts, x, Wroot, bconv.reshape(1, HID), Wh,
                   bh.reshape(1, 1))
    return out.reshape(N)


# speedup vs baseline: 6.1993x; 1.7424x over previous
import functools

import jax
import jax.numpy as jnp
from jax import lax
from jax.experimental import pallas as pl
from jax.experimental.pallas import tpu as pltpu
from jax.experimental.pallas import tpu_sc as plsc

N = 10000
E = 160000
D_IN = 128
HID = 16
K = 32
ROW = K * HID + HID
SROW = 32
NC = 2
NS = 16
NW = NC * NS
C = 64
CHUNKS = E // C
BASE_CH = 78
EXTRA_CH = 2
NEXTRA = (CHUNKS - BASE_CH * NW) // EXTRA_CH
MAXCH = BASE_CH + EXTRA_CH
ZR = 624
ZB = 208
MBLK = 1000


def _matmul_body(x_ref, w_ref, o_ref):
    o_ref[...] = jnp.dot(x_ref[...], w_ref[...],
                         preferred_element_type=jnp.float32)


def _precompute_q(x, w2aug):
    return pl.pallas_call(
        _matmul_body,
        grid=(N // MBLK,),
        in_specs=[
            pl.BlockSpec((MBLK, D_IN), lambda i: (i, 0)),
            pl.BlockSpec((D_IN, ROW), lambda i: (0, 0)),
        ],
        out_specs=pl.BlockSpec((MBLK, ROW), lambda i: (i, 0)),
        out_shape=jax.ShapeDtypeStruct((N, ROW), jnp.float32),
    )(x, w2aug)


def _combine_body(parts_ref, x_ref, wroot_ref, bconv_ref, wh_ref, bh_ref,
                  o_ref):
    s = parts_ref[0] + parts_ref[1]
    msg = s[:, 0:HID]
    cnt = s[:, HID:HID + 1]
    agg = msg / jnp.maximum(cnt, 1.0)
    root = jnp.dot(x_ref[...], wroot_ref[...],
                   preferred_element_type=jnp.float32)
    h = jnp.maximum(agg + root + bconv_ref[...], 0.0)
    o_ref[...] = jnp.dot(h, wh_ref[...],
                         preferred_element_type=jnp.float32) + bh_ref[0, 0]


def _combine(parts, x, wroot, bconv2, wh, bh2):
    return pl.pallas_call(
        _combine_body,
        grid=(N // MBLK,),
        in_specs=[
            pl.BlockSpec((NC, MBLK, SROW), lambda i: (0, i, 0)),
            pl.BlockSpec((MBLK, D_IN), lambda i: (i, 0)),
            pl.BlockSpec((D_IN, HID), lambda i: (0, 0)),
            pl.BlockSpec((1, HID), lambda i: (0, 0)),
            pl.BlockSpec((HID, 1), lambda i: (0, 0)),
            pl.BlockSpec((1, 1), lambda i: (0, 0)),
        ],
        out_specs=pl.BlockSpec((MBLK, 1), lambda i: (i, 0)),
        out_shape=jax.ShapeDtypeStruct((N, 1), jnp.float32),
    )(parts, x, wroot, bconv2, wh, bh2)


def _sc_body(q_hbm, src_hbm, dst_hbm, ew_hbm, w1_hbm, b1_hbm, out_hbm,
             srcb, dstb, ewb, qrows0, qrows1, msgb0, msgb1, zb, w1v, b1v,
             acc_sh, sem_g0, sem_g1, sem_s0, sem_s1):
    cid = lax.axis_index("c")
    sid = lax.axis_index("s")
    wid = sid * NC + cid

    qrows = (qrows0, qrows1)
    msgb = (msgb0, msgb1)
    sem_g = (sem_g0, sem_g1)
    sem_s = (sem_s0, sem_s1)

    zeros16 = jnp.zeros((HID,), jnp.float32)

    @pl.loop(0, ZB)
    def _zrow(r):
        zb[r, pl.ds(0, HID)] = zeros16
        zb[r, pl.ds(HID, HID)] = zeros16

    zstart = pl.multiple_of(sid * ZR, 8)

    @pl.loop(0, ZR // ZB)
    def _zcp(t):
        pltpu.sync_copy(zb, acc_sh.at[pl.ds(zstart + t * ZB, ZB)])

    @pl.when(sid == NS - 1)
    def _ztail():
        pltpu.sync_copy(zb.at[pl.ds(0, N - NS * ZR)],
                        acc_sh.at[pl.ds(NS * ZR, N - NS * ZR)])

    lane = lax.iota(jnp.int32, HID)
    cvec = jnp.where(lane == 0, 1.0, 0.0).astype(jnp.float32)

    @pl.loop(0, C)
    def _minit(e):
        msgb0[e, pl.ds(HID, HID)] = cvec
        msgb1[e, pl.ds(HID, HID)] = cvec

    pltpu.sync_copy(w1_hbm, w1v)
    pltpu.sync_copy(b1_hbm, b1v)
    w1lo, w1hi = w1v[pl.ds(0, 16)], w1v[pl.ds(16, 16)]
    b1lo, b1hi = b1v[pl.ds(0, 16)], b1v[pl.ds(16, 16)]
    w1s = [w1lo[i] for i in range(16)] + [w1hi[i] for i in range(16)]
    b1s = [b1lo[i] for i in range(16)] + [b1hi[i] for i in range(16)]

    nch = BASE_CH + jnp.where(wid < NEXTRA, EXTRA_CH, 0)
    ch0 = BASE_CH * wid + EXTRA_CH * jnp.minimum(wid, NEXTRA)
    base0 = jnp.minimum(ch0, CHUNKS - MAXCH)
    shift = ch0 - base0
    pltpu.sync_copy(src_hbm.at[pl.ds(base0, MAXCH)], srcb)
    pltpu.sync_copy(dst_hbm.at[pl.ds(base0, MAXCH)], dstb)
    pltpu.sync_copy(ew_hbm.at[pl.ds(base0, MAXCH)], ewb)

    plsc.subcore_barrier()

    def _gather(c, buf):
        return pltpu.async_copy(q_hbm.at[srcb.at[shift + c]], qrows[buf],
                                sem_g[buf])

    _gather(0, 0)

    def _chunk_step(c, b):
        @pl.when(c + 1 < nch)
        def _pref():
            _gather(c + 1, 1 - b)

        @pl.when(c >= 2)
        def _drain():
            pltpu.make_async_copy(msgb[b], acc_sh.at[dstb.at[shift + c - 2]],
                                  sem_s[b]).wait()

        pltpu.make_async_copy(q_hbm.at[srcb.at[shift + c]], qrows[b],
                              sem_g[b]).wait()

        jc = shift + c
        qr = qrows[b]
        mb = msgb[b]

        @pl.loop(0, C // 16)
        def _group(g):
            gb = g * 16
            ewv = ewb[jc, pl.ds(gb, 16)]
            hks = [jnp.maximum(ewv * w1s[k] + b1s[k], 0.0) for k in range(K)]
            for j in range(16):
                e = gb + j
                a = [qr[e, pl.ds(K * HID, HID)],
                     hks[0][j] * qr[e, pl.ds(0, HID)],
                     hks[1][j] * qr[e, pl.ds(HID, HID)],
                     hks[2][j] * qr[e, pl.ds(2 * HID, HID)]]
                for k in range(3, K):
                    a[(k + 1) % 4] = (a[(k + 1) % 4] + hks[k][j]
                                      * qr[e, pl.ds(k * HID, HID)])
                mb[e, pl.ds(0, HID)] = (a[0] + a[1]) + (a[2] + a[3])

        pltpu.async_copy(mb, acc_sh.at[dstb.at[jc]], sem_s[b], add=True)

    @pl.loop(0, BASE_CH // 2)
    def _pair(t):
        _chunk_step(2 * t, 0)
        _chunk_step(2 * t + 1, 1)

    @pl.when(nch > BASE_CH)
    def _extra_pair():
        _chunk_step(BASE_CH, 0)
        _chunk_step(BASE_CH + 1, 1)

    pltpu.make_async_copy(msgb[0], acc_sh.at[dstb.at[shift + nch - 2]],
                          sem_s[0]).wait()
    pltpu.make_async_copy(msgb[1], acc_sh.at[dstb.at[shift + nch - 1]],
                          sem_s[1]).wait()

    plsc.subcore_barrier()
    pltpu.sync_copy(acc_sh.at[pl.ds(zstart, ZR)],
                    out_hbm.at[cid, pl.ds(zstart, ZR)])

    @pl.when(sid == NS - 1)
    def _wtail():
        pltpu.sync_copy(acc_sh.at[pl.ds(NS * ZR, N - NS * ZR)],
                        out_hbm.at[cid, pl.ds(NS * ZR, N - NS * ZR)])


_sc_edge = functools.partial(
    pl.kernel,
    out_type=jax.ShapeDtypeStruct((NC, N, SROW), jnp.float32),
    mesh=plsc.VectorSubcoreMesh(core_axis_name="c", subcore_axis_name="s",
                                num_cores=NC, num_subcores=NS),
    compiler_params=pltpu.CompilerParams(use_tc_tiling_on_sc=False),
    scratch_types=[
        pltpu.VMEM((MAXCH, C), jnp.int32),
        pltpu.VMEM((MAXCH, C), jnp.int32),
        pltpu.VMEM((MAXCH, C), jnp.float32),
        pltpu.VMEM((C, ROW), jnp.float32),
        pltpu.VMEM((C, ROW), jnp.float32),
        pltpu.VMEM((C, SROW), jnp.float32),
        pltpu.VMEM((C, SROW), jnp.float32),
        pltpu.VMEM((ZB, SROW), jnp.float32),
        pltpu.VMEM((K,), jnp.float32),
        pltpu.VMEM((K,), jnp.float32),
        pltpu.VMEM_SHARED((N, SROW), jnp.float32),
        pltpu.SemaphoreType.DMA,
        pltpu.SemaphoreType.DMA,
        pltpu.SemaphoreType.DMA,
        pltpu.SemaphoreType.DMA,
    ],
)(_sc_body)


def kernel(x, edge_index, edge_weight, W1, b1, W2, b2, Wroot, bconv, Wh, bh):
    w2t = W2.reshape(K, D_IN, HID).transpose(1, 0, 2).reshape(D_IN, K * HID)
    w2aug = jnp.concatenate([w2t, b2.reshape(D_IN, HID)], axis=1)

    q = _precompute_q(x, w2aug)
    parts = _sc_edge(q, edge_index[0].reshape(CHUNKS, C),
                     edge_index[1].reshape(CHUNKS, C),
                     edge_weight.reshape(CHUNKS, C), W1.reshape(K), b1)
    out = _combine(parts, x, Wroot, bconv.reshape(1, HID), Wh,
                   bh.reshape(1, 1))
    return out.reshape(N)
